# Initial kernel scaffold; baseline (speedup 1.0000x reference)
import jax
import jax.numpy as jnp
from jax.experimental import pallas as pl


def kernel(hidden_states, synaptic_weights):
    raise NotImplementedError("write your pallas kernel here")


# speedup vs baseline: 289.8848x; 289.8848x over previous
import functools

import jax
import jax.numpy as jnp
from jax import lax
from jax.experimental import pallas as pl
from jax.experimental.pallas import tpu as pltpu
from jax.experimental.pallas import tpu_sc as plsc

_HIDDEN = 1024
_EXPERTS = 16
_THRESH = 1.0
_REFR_SET = 1.0
_LEAK = 0.9
_DT = 0.1

_MM_BM = 512
_SC_CHUNK = 512
_SM_BM = 512


def _matmul_body(x_ref, w_ref, out_ref):
    out_ref[...] = jnp.dot(x_ref[...], w_ref[...],
                           preferred_element_type=jnp.float32)


def _matmul(x, w):
    n, h = x.shape
    e = w.shape[1]
    grid = (n // _MM_BM,)
    return pl.pallas_call(
        _matmul_body,
        grid=grid,
        in_specs=[
            pl.BlockSpec((_MM_BM, h), lambda i: (i, 0)),
            pl.BlockSpec((h, e), lambda i: (0, 0)),
        ],
        out_specs=pl.BlockSpec((_MM_BM, e), lambda i: (i, 0)),
        out_shape=jax.ShapeDtypeStruct((n, e), jnp.float32),
    )(x, w)


def _scan_body(cur_hbm, mem_hbm, cur_v, out_v):
    wid = lax.axis_index("c") * 16 + lax.axis_index("s")
    n = cur_hbm.shape[0]
    nchunk = n // _SC_CHUNK

    @pl.when(wid == 0)
    def _():
        def step(t, carry):
            mem, refr = carry
            cur_t = cur_v[t]
            mem = mem * _LEAK
            active = refr <= 0.0
            mem = mem + jnp.where(active, cur_t, 0.0) * _DT
            refr = jnp.maximum(refr - _DT, 0.0)
            out_v[t] = mem
            spike = mem > _THRESH
            mem = jnp.where(spike, 0.0, mem)
            refr = jnp.where(spike, _REFR_SET, refr)
            return (mem, refr)

        mem = jnp.zeros((_EXPERTS,), jnp.float32)
        refr = jnp.zeros((_EXPERTS,), jnp.float32)
        for c in range(nchunk):
            base = c * _SC_CHUNK
            pltpu.sync_copy(cur_hbm.at[pl.ds(base, _SC_CHUNK)], cur_v)
            mem, refr = lax.fori_loop(0, _SC_CHUNK, step, (mem, refr))
            pltpu.sync_copy(out_v, mem_hbm.at[pl.ds(base, _SC_CHUNK)])


def _scan(cur):
    n, e = cur.shape
    mesh = plsc.VectorSubcoreMesh(core_axis_name="c", subcore_axis_name="s")
    return pl.kernel(
        _scan_body,
        out_type=jax.ShapeDtypeStruct((n, e), jnp.float32),
        mesh=mesh,
        scratch_types=[
            pltpu.VMEM((_SC_CHUNK, e), jnp.float32),
            pltpu.VMEM((_SC_CHUNK, e), jnp.float32),
        ],
    )(cur)


def _routing_body(mem_ref, out_ref):
    mem = mem_ref[...]
    spike = mem > _THRESH
    any_spike = jnp.any(spike, axis=-1, keepdims=True)
    sw = spike.astype(jnp.float32)
    r_spike = jax.nn.softmax(sw, axis=-1)
    r_nospike = jax.nn.softmax(mem / _THRESH, axis=-1)
    out_ref[...] = jnp.where(any_spike, r_spike, r_nospike)


def _routing(mem_pre):
    n, e = mem_pre.shape
    grid = (n // _SM_BM,)
    return pl.pallas_call(
        _routing_body,
        grid=grid,
        in_specs=[pl.BlockSpec((_SM_BM, e), lambda i: (i, 0))],
        out_specs=pl.BlockSpec((_SM_BM, e), lambda i: (i, 0)),
        out_shape=jax.ShapeDtypeStruct((n, e), jnp.float32),
    )(mem_pre)


@jax.jit
def kernel(hidden_states, synaptic_weights):
    b, s, h = hidden_states.shape
    e = synaptic_weights.shape[1]
    x = hidden_states.reshape(b * s, h)
    cur = _matmul(x, synaptic_weights)
    mem_pre = _scan(cur)
    routing = _routing(mem_pre)
    return routing.reshape(b, s, e)


# speedup vs baseline: 414.1199x; 1.4286x over previous
import jax
import jax.numpy as jnp
from jax import lax
from jax.experimental import pallas as pl
from jax.experimental.pallas import tpu as pltpu
from jax.experimental.pallas import tpu_sc as plsc

_EXPERTS = 16
_THRESH = 1.0
_LEAK = 0.9
_DT = 0.1
_RC = 10

_MM_BM = 512
_SC_CHUNK = 1024
_UNROLL = 4
_SM_BM = 512


def _matmul_body(x_ref, w_ref, out_ref):
    out_ref[...] = jnp.dot(x_ref[...], w_ref[...],
                           preferred_element_type=jnp.float32) * _DT


def _matmul(x, w):
    n, h = x.shape
    e = w.shape[1]
    return pl.pallas_call(
        _matmul_body,
        grid=(n // _MM_BM,),
        in_specs=[
            pl.BlockSpec((_MM_BM, h), lambda i: (i, 0)),
            pl.BlockSpec((h, e), lambda i: (0, 0)),
        ],
        out_specs=pl.BlockSpec((_MM_BM, e), lambda i: (i, 0)),
        out_shape=jax.ShapeDtypeStruct((n, e), jnp.float32),
    )(x, w)


def _scan_body(cdt_hbm, mem_hbm, cur_a, cur_b, out_v, sem_a, sem_b):
    wid = lax.axis_index("c") * 16 + lax.axis_index("s")
    n = cdt_hbm.shape[0] // _EXPERTS
    nch = n // _SC_CHUNK
    cw = _SC_CHUNK * _EXPERTS

    @pl.when(wid == 0)
    def _():
        bufs = (cur_a, cur_b)
        sems = (sem_a, sem_b)
        cps = [None] * nch
        for c in range(min(2, nch)):
            cps[c] = pltpu.async_copy(
                cdt_hbm.at[pl.ds(c * cw, cw)], bufs[c], sems[c])

        mp = jnp.zeros((_EXPERTS,), jnp.float32)
        rc = jnp.zeros((_EXPERTS,), jnp.int32)
        for c in range(nch):
            buf = bufs[c % 2]
            cps[c].wait()
            base = c * _SC_CHUNK

            def body(i, carry, buf=buf, base=base):
                mp, rc = carry
                for j in range(_UNROLL):
                    t = i * _UNROLL + j
                    spike = mp > _THRESH
                    pre = rc <= 1
                    cg = jnp.where(pre, buf[pl.ds(t * _EXPERTS, _EXPERTS)], 0.0)
                    m9 = jnp.where(spike, 0.0, mp * _LEAK)
                    mp = jnp.where(spike, m9, m9 + cg)
                    out_v[pl.ds((base + t) * _EXPERTS, _EXPERTS)] = mp
                    rc = jnp.where(spike, _RC, rc - 1)
                return (mp, rc)

            mp, rc = lax.fori_loop(0, _SC_CHUNK // _UNROLL, body, (mp, rc))
            if c + 2 < nch:
                cps[c + 2] = pltpu.async_copy(
                    cdt_hbm.at[pl.ds((c + 2) * cw, cw)], buf, sems[c % 2])

        pltpu.sync_copy(out_v, mem_hbm)


def _scan(cdt):
    n, e = cdt.shape
    mesh = plsc.VectorSubcoreMesh(core_axis_name="c", subcore_axis_name="s")
    flat = cdt.reshape(n * e)
    out = pl.kernel(
        _scan_body,
        out_type=jax.ShapeDtypeStruct((n * e,), jnp.float32),
        mesh=mesh,
        scratch_types=[
            pltpu.VMEM((_SC_CHUNK * _EXPERTS,), jnp.float32),
            pltpu.VMEM((_SC_CHUNK * _EXPERTS,), jnp.float32),
            pltpu.VMEM((n * e,), jnp.float32),
            pltpu.SemaphoreType.DMA,
            pltpu.SemaphoreType.DMA,
        ],
    )(flat)
    return out.reshape(n, e)


def _routing_body(mem_ref, out_ref):
    mem = mem_ref[...]
    spike = mem > _THRESH
    any_spike = jnp.any(spike, axis=-1, keepdims=True)
    sel = jnp.where(any_spike, spike.astype(jnp.float32), mem / _THRESH)
    out_ref[...] = jax.nn.softmax(sel, axis=-1)


def _routing(mem_pre):
    n, e = mem_pre.shape
    return pl.pallas_call(
        _routing_body,
        grid=(n // _SM_BM,),
        in_specs=[pl.BlockSpec((_SM_BM, e), lambda i: (i, 0))],
        out_specs=pl.BlockSpec((_SM_BM, e), lambda i: (i, 0)),
        out_shape=jax.ShapeDtypeStruct((n, e), jnp.float32),
    )(mem_pre)


@jax.jit
def kernel(hidden_states, synaptic_weights):
    b, s, h = hidden_states.shape
    e = synaptic_weights.shape[1]
    x = hidden_states.reshape(b * s, h)
    cdt = _matmul(x, synaptic_weights)
    mem_pre = _scan(cdt)
    routing = _routing(mem_pre)
    return routing.reshape(b, s, e)


# speedup vs baseline: 496.1517x; 1.1981x over previous
import jax
import jax.numpy as jnp
from jax import lax
from jax.experimental import pallas as pl
from jax.experimental.pallas import tpu as pltpu
from jax.experimental.pallas import tpu_sc as plsc

_EXPERTS = 16
_THRESH = 1.0
_LEAK = 0.9
_DT = 0.1
_RC = 10

_MM_BM = 512
_SC_CHUNK = 1024
_UNROLL = 8
_SM_BM = 512


def _matmul_body(x_ref, w_ref, out_ref):
    out_ref[...] = jnp.dot(x_ref[...], w_ref[...],
                           preferred_element_type=jnp.float32) * _DT


def _matmul(x, w):
    n, h = x.shape
    e = w.shape[1]
    return pl.pallas_call(
        _matmul_body,
        grid=(n // _MM_BM,),
        in_specs=[
            pl.BlockSpec((_MM_BM, h), lambda i: (i, 0)),
            pl.BlockSpec((h, e), lambda i: (0, 0)),
        ],
        out_specs=pl.BlockSpec((_MM_BM, e), lambda i: (i, 0)),
        out_shape=jax.ShapeDtypeStruct((n, e), jnp.float32),
    )(x, w)


def _scan_body(cdt_hbm, mem_hbm, cur_a, cur_b, out_v, sem_a, sem_b):
    wid = lax.axis_index("c") * 16 + lax.axis_index("s")
    n = cdt_hbm.shape[0] // _EXPERTS
    nch = n // _SC_CHUNK
    cw = _SC_CHUNK * _EXPERTS

    @pl.when(wid == 0)
    def _():
        bufs = (cur_a, cur_b)
        sems = (sem_a, sem_b)
        cps = [None] * nch
        for c in range(min(2, nch)):
            cps[c] = pltpu.async_copy(
                cdt_hbm.at[pl.ds(c * cw, cw)], bufs[c].at[pl.ds(0, cw)],
                sems[c])

        mp = jnp.zeros((_EXPERTS,), jnp.float32)
        rc = jnp.zeros((_EXPERTS,), jnp.int32)
        for c in range(nch):
            buf = bufs[c % 2]
            cps[c].wait()
            base = c * _SC_CHUNK
            cg = jnp.where(rc <= 1, buf[pl.ds(0, _EXPERTS)], 0.0)

            def body(i, carry, buf=buf, base=base):
                mp, rc, cg = carry
                for j in range(_UNROLL):
                    t = i * _UNROLL + j
                    spike = mp > _THRESH
                    mp = jnp.where(spike, 0.0, mp * _LEAK + cg)
                    out_v[pl.ds((base + t) * _EXPERTS, _EXPERTS)] = mp
                    rc = jnp.where(spike, _RC, rc - 1)
                    cg = jnp.where(rc <= 1,
                                   buf[pl.ds((t + 1) * _EXPERTS, _EXPERTS)],
                                   0.0)
                return (mp, rc, cg)

            mp, rc, cg = lax.fori_loop(0, _SC_CHUNK // _UNROLL, body,
                                       (mp, rc, cg))
            if c + 2 < nch:
                cps[c + 2] = pltpu.async_copy(
                    cdt_hbm.at[pl.ds((c + 2) * cw, cw)],
                    buf.at[pl.ds(0, cw)], sems[c % 2])

        pltpu.sync_copy(out_v, mem_hbm)


def _scan(cdt):
    n, e = cdt.shape
    mesh = plsc.VectorSubcoreMesh(core_axis_name="c", subcore_axis_name="s")
    flat = cdt.reshape(n * e)
    out = pl.kernel(
        _scan_body,
        out_type=jax.ShapeDtypeStruct((n * e,), jnp.float32),
        mesh=mesh,
        scratch_types=[
            pltpu.VMEM((_SC_CHUNK * _EXPERTS + _EXPERTS,), jnp.float32),
            pltpu.VMEM((_SC_CHUNK * _EXPERTS + _EXPERTS,), jnp.float32),
            pltpu.VMEM((n * e,), jnp.float32),
            pltpu.SemaphoreType.DMA,
            pltpu.SemaphoreType.DMA,
        ],
    )(flat)
    return out.reshape(n, e)


def _routing_body(mem_ref, out_ref):
    mem = mem_ref[...]
    spike = mem > _THRESH
    any_spike = jnp.any(spike, axis=-1, keepdims=True)
    sel = jnp.where(any_spike, spike.astype(jnp.float32), mem / _THRESH)
    out_ref[...] = jax.nn.softmax(sel, axis=-1)


def _routing(mem_pre):
    n, e = mem_pre.shape
    return pl.pallas_call(
        _routing_body,
        grid=(n // _SM_BM,),
        in_specs=[pl.BlockSpec((_SM_BM, e), lambda i: (i, 0))],
        out_specs=pl.BlockSpec((_SM_BM, e), lambda i: (i, 0)),
        out_shape=jax.ShapeDtypeStruct((n, e), jnp.float32),
    )(mem_pre)


@jax.jit
def kernel(hidden_states, synaptic_weights):
    b, s, h = hidden_states.shape
    e = synaptic_weights.shape[1]
    x = hidden_states.reshape(b * s, h)
    cdt = _matmul(x, synaptic_weights)
    mem_pre = _scan(cdt)
    routing = _routing(mem_pre)
    return routing.reshape(b, s, e)
